# baseline (speedup 1.0000x reference)
import functools

import jax
import jax.numpy as jnp
from jax import lax
from jax.experimental import pallas as pl
from jax.experimental.pallas import tpu as pltpu
from jax.experimental.pallas import tpu_sc as plsc

_B, _E, _S, _D = 16, 20, 2048, 32
_ES = _E * _S
_CHUNKS2 = _E * (_S // 2) // 32
_BPC = _B // 2
_BSTRIDE = _E * _D * _S
_NIDX = 16 * _S


def _shift_up(v, ids):
    return v.at[ids].get(mode="promise_in_bounds")


def _body(h_hbm, pos_hbm, out_hbm, pos_v, table_v, lo_v, hi_v,
          validf_v, idx_v, vals_v, shared_t, sem):
    c = lax.axis_index("c")
    s = lax.axis_index("s")
    b = c * _BPC + s // 2
    half = s % 2
    iot = lax.iota(jnp.int32, 16)

    pltpu.sync_copy(pos_hbm.at[:, b, pl.ds(half * (_S // 2), _S // 2)],
                    pos_v)

    neg1 = jnp.full((16,), -1, jnp.int32)
    for k in range(_S // 16):
        table_v[pl.ds(k * 16, 16)] = neg1

    i0 = half * (_S // 2)
    ids = jnp.minimum(iot + 1, 15)
    lane15 = iot == 15

    def chunk_body(j, carry):
        for u in range(4):
            jj = j * 4 + u
            idx = pos_v[jj >> 6, pl.ds((jj & 63) * 16, 16)]
            packed = idx * 65536 + (jj * 16 + (jj >> 6) * 1024 + i0 + iot)
            srt, _ = plsc.sort_key_val(packed, packed)
            spos = srt >> 16
            sval = srt & 0xFFFF
            last = (spos != _shift_up(spos, ids)) | lane15
            plsc.store_scatter(table_v, [spos], sval, mask=last)
        return carry

    lax.fori_loop(0, _CHUNKS2 // 2, chunk_body, jnp.int32(0))

    pltpu.sync_copy(table_v, shared_t.at[s])
    plsc.subcore_barrier()
    pltpu.sync_copy(shared_t.at[(s // 2) * 2], lo_v)
    pltpu.sync_copy(shared_t.at[(s // 2) * 2 + 1], hi_v)

    bbase = b * _BSTRIDE

    def merge_body(g, n_inv):
        t = jnp.maximum(lo_v[pl.ds(g * 16, 16)], hi_v[pl.ds(g * 16, 16)])
        tc = jnp.maximum(t, 0)
        sw = tc & (_S - 1)
        base = (bbase + (tc >> 11) * (_D * _S)
                + (sw >> 7) * 1024 + (sw & 127))
        inv = t < 0
        validf_v[pl.ds(g * 16, 16)] = jnp.where(inv, jnp.float32(0.0),
                                                jnp.float32(1.0))
        tb = (g >> 3) * 1024 + (g & 7) * 16
        for dhl in range(2):
            for dlo in range(8):
                dphys = (half * 2 + dhl) * (_D * _S // 4) + dlo * 128
                idx_v[pl.ds(tb + dhl * 16384 + dlo * 128, 16)] = base + dphys
        return n_inv + jnp.sum(inv.astype(jnp.int32))

    n_inv = lax.fori_loop(0, _S // 16, merge_body, jnp.int32(0))

    pltpu.async_copy(h_hbm.at[idx_v], vals_v, sem).wait()

    @pl.when(n_inv > 0)
    def _fix():
        def fix_body(v, carry):
            m = validf_v[pl.ds(((v >> 6) & 15) * 128 + (v & 7) * 16, 16)]
            vals_v[pl.ds(v * 16, 16)] = vals_v[pl.ds(v * 16, 16)] * m
            return carry

        lax.fori_loop(0, _NIDX // 16, fix_body, jnp.int32(0))

    for dhl in range(2):
        pltpu.sync_copy(
            vals_v.at[pl.ds(dhl * 16384, 16384)],
            out_hbm.at[pl.ds(b * (_D * _S) + (half * 2 + dhl) * 16384,
                             16384)])


@jax.jit
def _realign(h_flat, pos_t):
    mesh = plsc.VectorSubcoreMesh(core_axis_name="c", subcore_axis_name="s")
    return pl.kernel(
        _body,
        mesh=mesh,
        compiler_params=pltpu.CompilerParams(needs_layout_passes=False),
        out_type=jax.ShapeDtypeStruct((_B * _D * _S,), jnp.float32),
        scratch_types=[
            pltpu.VMEM((_E, _S // 2), jnp.int32),
            pltpu.VMEM((_S,), jnp.int32),
            pltpu.VMEM((_S,), jnp.int32),
            pltpu.VMEM((_S,), jnp.int32),
            pltpu.VMEM((_S,), jnp.float32),
            pltpu.VMEM((_NIDX,), jnp.int32),
            pltpu.VMEM((_NIDX,), jnp.float32),
            pltpu.VMEM_SHARED((16, _S), jnp.int32),
            pltpu.SemaphoreType.DMA,
        ],
    )(h_flat, pos_t)


def kernel(history_embedding_multivariate, seq_positions_multivariate,
           seq_length):
    del seq_length
    h = history_embedding_multivariate
    B, E, S, D = h.shape
    h_flat = (h.reshape(B, E, S // 128, 128, D // 8, 8)
              .transpose(0, 1, 4, 2, 5, 3).reshape(B * E * D * S))
    pos_t = seq_positions_multivariate.astype(jnp.int32).transpose(1, 0, 2)
    out = _realign(h_flat, pos_t)
    return (out.reshape(B, D // 8, S // 128, 8, 128)
            .transpose(0, 2, 4, 1, 3).reshape(B, S, D))


# speedup vs baseline: 1.2774x; 1.2774x over previous
import functools

import jax
import jax.numpy as jnp
from jax import lax
from jax.experimental import pallas as pl
from jax.experimental.pallas import tpu as pltpu
from jax.experimental.pallas import tpu_sc as plsc

_B, _E, _S, _D = 16, 20, 2048, 32
_ES = _E * _S
_CHUNKS2 = _E * (_S // 2) // 32
_BPC = _B // 2
_BSTRIDE = _E * _D * _S
_NIDX = 16 * _S


def _shift_up(v, ids):
    return v.at[ids].get(mode="promise_in_bounds")


def _body(h_hbm, pos_hbm, out_hbm, pos_v, table_v, lo_v, hi_v,
          validf_v, idx_v, vals_v, shared_t, sem):
    c = lax.axis_index("c")
    s = lax.axis_index("s")
    b = c * _BPC + s // 2
    half = s % 2
    iot = lax.iota(jnp.int32, 16)

    pltpu.sync_copy(pos_hbm.at[:, b, pl.ds(half * (_S // 2), _S // 2)],
                    pos_v)

    neg1 = jnp.full((16,), -1, jnp.int32)
    for k in range(_S // 16):
        table_v[pl.ds(k * 16, 16)] = neg1

    i0 = half * (_S // 2)
    ids = jnp.minimum(iot + 1, 15)
    lane15 = iot == 15

    def chunk_body(j, carry):
        for u in range(4):
            jj = j * 4 + u
            idx = pos_v[jj >> 6, pl.ds((jj & 63) * 16, 16)]
            ival = jj * 16 + (jj >> 6) * 1024 + i0 + iot
            plsc.store_scatter(table_v, [idx], ival)
        return carry

    lax.fori_loop(0, _CHUNKS2 // 2, chunk_body, jnp.int32(0))

    pltpu.sync_copy(table_v, shared_t.at[s])
    plsc.subcore_barrier()
    pltpu.sync_copy(shared_t.at[(s // 2) * 2], lo_v)
    pltpu.sync_copy(shared_t.at[(s // 2) * 2 + 1], hi_v)

    bbase = b * _BSTRIDE

    def merge_body(g, n_inv):
        t = jnp.maximum(lo_v[pl.ds(g * 16, 16)], hi_v[pl.ds(g * 16, 16)])
        tc = jnp.maximum(t, 0)
        sw = tc & (_S - 1)
        base = (bbase + (tc >> 11) * (_D * _S)
                + (sw >> 7) * 1024 + (sw & 127))
        inv = t < 0
        validf_v[pl.ds(g * 16, 16)] = jnp.where(inv, jnp.float32(0.0),
                                                jnp.float32(1.0))
        tb = (g >> 3) * 1024 + (g & 7) * 16
        for dhl in range(2):
            for dlo in range(8):
                dphys = (half * 2 + dhl) * (_D * _S // 4) + dlo * 128
                idx_v[pl.ds(tb + dhl * 16384 + dlo * 128, 16)] = base + dphys
        return n_inv + jnp.sum(inv.astype(jnp.int32))

    n_inv = lax.fori_loop(0, _S // 16, merge_body, jnp.int32(0))

    pltpu.async_copy(h_hbm.at[idx_v], vals_v, sem).wait()

    @pl.when(n_inv > 0)
    def _fix():
        def fix_body(v, carry):
            m = validf_v[pl.ds(((v >> 6) & 15) * 128 + (v & 7) * 16, 16)]
            vals_v[pl.ds(v * 16, 16)] = vals_v[pl.ds(v * 16, 16)] * m
            return carry

        lax.fori_loop(0, _NIDX // 16, fix_body, jnp.int32(0))

    for dhl in range(2):
        pltpu.sync_copy(
            vals_v.at[pl.ds(dhl * 16384, 16384)],
            out_hbm.at[pl.ds(b * (_D * _S) + (half * 2 + dhl) * 16384,
                             16384)])


@jax.jit
def _realign(h_flat, pos_t):
    mesh = plsc.VectorSubcoreMesh(core_axis_name="c", subcore_axis_name="s")
    return pl.kernel(
        _body,
        mesh=mesh,
        compiler_params=pltpu.CompilerParams(needs_layout_passes=False),
        out_type=jax.ShapeDtypeStruct((_B * _D * _S,), jnp.float32),
        scratch_types=[
            pltpu.VMEM((_E, _S // 2), jnp.int32),
            pltpu.VMEM((_S,), jnp.int32),
            pltpu.VMEM((_S,), jnp.int32),
            pltpu.VMEM((_S,), jnp.int32),
            pltpu.VMEM((_S,), jnp.float32),
            pltpu.VMEM((_NIDX,), jnp.int32),
            pltpu.VMEM((_NIDX,), jnp.float32),
            pltpu.VMEM_SHARED((16, _S), jnp.int32),
            pltpu.SemaphoreType.DMA,
        ],
    )(h_flat, pos_t)


def kernel(history_embedding_multivariate, seq_positions_multivariate,
           seq_length):
    del seq_length
    h = history_embedding_multivariate
    B, E, S, D = h.shape
    h_flat = (h.reshape(B, E, S // 128, 128, D // 8, 8)
              .transpose(0, 1, 4, 2, 5, 3).reshape(B * E * D * S))
    pos_t = seq_positions_multivariate.astype(jnp.int32).transpose(1, 0, 2)
    out = _realign(h_flat, pos_t)
    return (out.reshape(B, D // 8, S // 128, 8, 128)
            .transpose(0, 2, 4, 1, 3).reshape(B, S, D))


# speedup vs baseline: 1.2805x; 1.0024x over previous
import functools

import jax
import jax.numpy as jnp
from jax import lax
from jax.experimental import pallas as pl
from jax.experimental.pallas import tpu as pltpu
from jax.experimental.pallas import tpu_sc as plsc

_B, _E, _S, _D = 16, 20, 2048, 32
_ES = _E * _S
_CHUNKS2 = _E * (_S // 2) // 32
_BPC = _B // 2
_BSTRIDE = _E * _D * _S
_NIDX = 16 * _S


def _shift_up(v, ids):
    return v.at[ids].get(mode="promise_in_bounds")


def _body(h_hbm, pos_hbm, out_hbm, pos_v, table_v, lo_v, hi_v,
          validf_v, idx_v, vals_v, shared_t, sem):
    c = lax.axis_index("c")
    s = lax.axis_index("s")
    b = c * _BPC + s // 2
    half = s % 2
    iot = lax.iota(jnp.int32, 16)

    pltpu.sync_copy(pos_hbm.at[:, b, pl.ds(half * (_S // 2), _S // 2)],
                    pos_v)

    neg1 = jnp.full((16,), -1, jnp.int32)
    for k in range(_S // 16):
        table_v[pl.ds(k * 16, 16)] = neg1

    i0 = half * (_S // 2)
    ids = jnp.minimum(iot + 1, 15)
    lane15 = iot == 15

    def chunk_body(j, carry):
        for u in range(4):
            jj = j * 4 + u
            idx = pos_v[jj >> 6, pl.ds((jj & 63) * 16, 16)]
            ival = jj * 16 + (jj >> 6) * 1024 + i0 + iot
            plsc.store_scatter(table_v, [idx], ival)
        return carry

    lax.fori_loop(0, _CHUNKS2 // 2, chunk_body, jnp.int32(0))

    pltpu.sync_copy(table_v, shared_t.at[s])
    plsc.subcore_barrier()
    pltpu.sync_copy(shared_t.at[(s // 2) * 2], lo_v)
    pltpu.sync_copy(shared_t.at[(s // 2) * 2 + 1], hi_v)

    bbase = b * _BSTRIDE

    def merge_body(g, n_inv):
        t = jnp.maximum(lo_v[pl.ds(g * 16, 16)], hi_v[pl.ds(g * 16, 16)])
        tc = jnp.maximum(t, 0)
        sw = tc & (_S - 1)
        base = (bbase + (tc >> 11) * (_D * _S)
                + (sw >> 7) * 1024 + (sw & 127))
        inv = t < 0
        validf_v[pl.ds(g * 16, 16)] = jnp.where(inv, jnp.float32(0.0),
                                                jnp.float32(1.0))
        tb = (g >> 3) * 1024 + (g & 7) * 16
        for dhl in range(2):
            for dlo in range(8):
                dphys = (half * 2 + dhl) * (_D * _S // 4) + dlo * 128
                idx_v[pl.ds(tb + dhl * 16384 + dlo * 128, 16)] = base + dphys
        return n_inv + jnp.sum(inv.astype(jnp.int32))

    n_inv = lax.fori_loop(0, _S // 16, merge_body, jnp.int32(0))

    pltpu.async_copy(h_hbm.at[idx_v], vals_v, sem).wait()

    @pl.when(n_inv > 0)
    def _fix():
        def fix_body(v, carry):
            m = validf_v[pl.ds(((v >> 6) & 15) * 128 + (v & 7) * 16, 16)]
            vals_v[pl.ds(v * 16, 16)] = vals_v[pl.ds(v * 16, 16)] * m
            return carry

        lax.fori_loop(0, _NIDX // 16, fix_body, jnp.int32(0))

    for dhl in range(2):
        pltpu.sync_copy(
            vals_v.at[pl.ds(dhl * 16384, 16384)],
            out_hbm.at[pl.ds(b * (_D * _S) + (half * 2 + dhl) * 16384,
                             16384)])


@jax.jit
def _realign(h_flat, pos_t):
    mesh = plsc.VectorSubcoreMesh(core_axis_name="c", subcore_axis_name="s")
    return pl.kernel(
        _body,
        mesh=mesh,
        compiler_params=pltpu.CompilerParams(needs_layout_passes=False,
                                             disable_bounds_checks=True),
        out_type=jax.ShapeDtypeStruct((_B * _D * _S,), jnp.float32),
        scratch_types=[
            pltpu.VMEM((_E, _S // 2), jnp.int32),
            pltpu.VMEM((_S,), jnp.int32),
            pltpu.VMEM((_S,), jnp.int32),
            pltpu.VMEM((_S,), jnp.int32),
            pltpu.VMEM((_S,), jnp.float32),
            pltpu.VMEM((_NIDX,), jnp.int32),
            pltpu.VMEM((_NIDX,), jnp.float32),
            pltpu.VMEM_SHARED((16, _S), jnp.int32),
            pltpu.SemaphoreType.DMA,
        ],
    )(h_flat, pos_t)


def kernel(history_embedding_multivariate, seq_positions_multivariate,
           seq_length):
    del seq_length
    h = history_embedding_multivariate
    B, E, S, D = h.shape
    h_flat = (h.reshape(B, E, S // 128, 128, D // 8, 8)
              .transpose(0, 1, 4, 2, 5, 3).reshape(B * E * D * S))
    pos_t = seq_positions_multivariate.astype(jnp.int32).transpose(1, 0, 2)
    out = _realign(h_flat, pos_t)
    return (out.reshape(B, D // 8, S // 128, 8, 128)
            .transpose(0, 2, 4, 1, 3).reshape(B, S, D))
